# baseline (speedup 1.0000x reference)
import jax
import jax.numpy as jnp
from jax import lax
from jax.experimental import pallas as pl
from jax.experimental.pallas import tpu as pltpu
from jax.experimental.pallas import tpu_sc as plsc

NC = 2
NS = 16
NW = NC * NS
D = 64
CHUNK = 128
SUB = 4
STAGE = CHUNK * SUB


def _gather_body(table_hbm, idx_hbm, out_hbm,
                 idx_v, rows_a, rows_b, sem_a, sem_b, sem_sa, sem_sb):
    wid = lax.axis_index("s") * NC + lax.axis_index("c")
    chunks_per_w = idx_v.shape[0]
    n_per_w = chunks_per_w * CHUNK
    n_stages = chunks_per_w // SUB
    base = wid * n_per_w

    pltpu.sync_copy(idx_hbm.at[pl.ds(wid * chunks_per_w, chunks_per_w)], idx_v)

    def fire(stage, rows_v, sem):
        for j in range(SUB):
            pltpu.async_copy(
                table_hbm.at[idx_v.at[stage * SUB + j]],
                rows_v.at[pl.ds(j * CHUNK, CHUNK)],
                sem,
            )

    def wait_gathers(stage, rows_v, sem):
        for j in range(SUB):
            pltpu.make_async_copy(
                table_hbm.at[idx_v.at[stage * SUB + j]],
                rows_v.at[pl.ds(j * CHUNK, CHUNK)],
                sem,
            ).wait()

    def store(stage, rows_v, sem):
        pltpu.async_copy(
            rows_v, out_hbm.at[pl.ds(base + stage * STAGE, STAGE)], sem
        )

    def wait_store(stage, rows_v, sem):
        pltpu.make_async_copy(
            rows_v, out_hbm.at[pl.ds(base + stage * STAGE, STAGE)], sem
        ).wait()

    fire(0, rows_a, sem_a)
    fire(1, rows_b, sem_b)
    wait_gathers(0, rows_a, sem_a)
    store(0, rows_a, sem_sa)

    def one_stage(s, cur, cur_gsem, cur_ssem, other, other_gsem, other_ssem):
        wait_store(s - 1, other, other_ssem)

        @pl.when(s + 1 < n_stages)
        def _():
            fire(s + 1, other, other_gsem)

        wait_gathers(s, cur, cur_gsem)
        store(s, cur, cur_ssem)

    def pair(k, carry):
        s = 2 * k + 1
        one_stage(s, rows_b, sem_b, sem_sb, rows_a, sem_a, sem_sa)
        one_stage(s + 1, rows_a, sem_a, sem_sa, rows_b, sem_b, sem_sb)
        return carry

    lax.fori_loop(0, (n_stages - 2) // 2, pair, 0, unroll=False)
    s_last = n_stages - 1
    one_stage(s_last, rows_b, sem_b, sem_sb, rows_a, sem_a, sem_sa)
    wait_store(s_last, rows_b, sem_sb)


def _sc_gather(table, idx2d):
    n_chunks = idx2d.shape[0]
    n = n_chunks * CHUNK
    chunks_per_w = n_chunks // NW
    mesh = plsc.VectorSubcoreMesh(
        core_axis_name="c", subcore_axis_name="s", num_cores=NC, num_subcores=NS
    )
    return pl.kernel(
        _gather_body,
        out_type=jax.ShapeDtypeStruct((n, D), jnp.float32),
        mesh=mesh,
        scratch_types=[
            pltpu.VMEM((chunks_per_w, CHUNK), jnp.int32),
            pltpu.VMEM((STAGE, D), jnp.float32),
            pltpu.VMEM((STAGE, D), jnp.float32),
            pltpu.SemaphoreType.DMA,
            pltpu.SemaphoreType.DMA,
            pltpu.SemaphoreType.DMA,
            pltpu.SemaphoreType.DMA,
        ],
        compiler_params=pltpu.CompilerParams(use_tc_tiling_on_sc=False),
        name="sc_embedding_gather",
    )(table, idx2d)


def _dense_body(x_ref, w_ref, b_ref, o_ref):
    acc = jnp.dot(x_ref[...], w_ref[...], preferred_element_type=jnp.float32)
    o_ref[...] = jnp.tanh(acc + b_ref[...])


def _dense(x, W, b):
    n = x.shape[0]
    blk = 8192
    return pl.pallas_call(
        _dense_body,
        grid=(n // blk,),
        in_specs=[
            pl.BlockSpec((blk, D), lambda i: (i, 0)),
            pl.BlockSpec((D, D), lambda i: (0, 0)),
            pl.BlockSpec((1, D), lambda i: (0, 0)),
        ],
        out_specs=pl.BlockSpec((blk, D), lambda i: (i, 0)),
        out_shape=jax.ShapeDtypeStruct((n, D), jnp.float32),
        name="dense_tanh",
    )(x, W, b.reshape(1, D))


def kernel(inputs, table, W, b):
    B, L = inputs.shape
    idx2d = inputs.reshape(-1, CHUNK).astype(jnp.int32)
    gathered = _sc_gather(table, idx2d)
    out = _dense(gathered, W, b)
    return out.reshape(B, L, D)
